# baseline (speedup 1.0000x reference)
import jax
import jax.numpy as jnp
from jax.experimental import pallas as pl
from jax.experimental.pallas import tpu as pltpu

BLK = 4096


def _body(x_ref, w_ref, b_ref, dv_ref, skeys_ref, starts_ref, o_ref):
    k = pl.program_id(0)

    def fix(s, carry):
        e = skeys_ref[s]
        local = (e >> 11) - k * BLK
        src = e & (2 ** 11 - 1)
        w_ref[pl.ds(local, 1), :] = dv_ref[pl.ds(src, 1), :]
        return carry

    jax.lax.fori_loop(starts_ref[k], starts_ref[k + 1], fix, 0)

    acc = jax.lax.dot_general(
        x_ref[...], w_ref[...], (((1,), (1,)), ((), ())),
        preferred_element_type=jnp.float32)
    o_ref[...] = jnp.maximum(acc + b_ref[...], 0.0)


def kernel(x, dict_idx, dict_val, W, b):
    L, F = W.shape
    B = x.shape[0]
    U = dict_idx.shape[0]
    nblk = L // BLK

    keys = (dict_idx.astype(jnp.int32) << 11) | jnp.arange(U, dtype=jnp.int32)
    skeys = jax.lax.sort(keys)
    bounds = (jnp.arange(0, L + 1, BLK, dtype=jnp.int32)) << 11
    starts = jnp.searchsorted(skeys, bounds, side="left").astype(jnp.int32)

    b2 = b.reshape(1, L)

    out = pl.pallas_call(
        _body,
        grid=(nblk,),
        in_specs=[
            pl.BlockSpec((B, F), lambda k: (0, 0)),
            pl.BlockSpec((BLK, F), lambda k: (k, 0)),
            pl.BlockSpec((1, BLK), lambda k: (0, k)),
            pl.BlockSpec((U, F), lambda k: (0, 0)),
            pl.BlockSpec(memory_space=pltpu.SMEM),
            pl.BlockSpec(memory_space=pltpu.SMEM),
        ],
        out_specs=pl.BlockSpec((B, BLK), lambda k: (0, k)),
        out_shape=jax.ShapeDtypeStruct((B, L), jnp.float32),
    )(x, W, b2, dict_val, skeys, starts)
    return out


# speedup vs baseline: 1.0224x; 1.0224x over previous
import jax
import jax.numpy as jnp
from jax.experimental import pallas as pl
from jax.experimental.pallas import tpu as pltpu

BLK = 4096


def _lower_bound(skeys_ref, u, target):
    def step(_, lohi):
        lo, hi = lohi
        mid = jnp.minimum((lo + hi) // 2, u - 1)
        pred = jnp.logical_and(lo < hi, skeys_ref[mid] < target)
        lo2 = jnp.where(pred, mid + 1, lo)
        hi2 = jnp.where(jnp.logical_and(lo < hi, jnp.logical_not(pred)),
                        mid, hi)
        return lo2, hi2

    lo, _ = jax.lax.fori_loop(0, 12, step, (0, u))
    return lo


def _body(x_ref, w_ref, b_ref, dv_ref, skeys_ref, o_ref):
    k = pl.program_id(0)
    u = skeys_ref.shape[0]

    def fix(s, carry):
        e = skeys_ref[s]
        local = (e >> 11) - k * BLK
        src = e & (2 ** 11 - 1)
        w_ref[pl.ds(local, 1), :] = dv_ref[pl.ds(src, 1), :]
        return carry

    s0 = _lower_bound(skeys_ref, u, (k * BLK) << 11)
    s1 = _lower_bound(skeys_ref, u, ((k + 1) * BLK) << 11)
    jax.lax.fori_loop(s0, s1, fix, 0)

    acc = jax.lax.dot_general(
        x_ref[...], w_ref[...], (((1,), (1,)), ((), ())),
        preferred_element_type=jnp.float32)
    o_ref[...] = jnp.maximum(acc + b_ref[...], 0.0)


def kernel(x, dict_idx, dict_val, W, b):
    L, F = W.shape
    B = x.shape[0]
    U = dict_idx.shape[0]
    nblk = L // BLK

    keys = (dict_idx.astype(jnp.int32) << 11) | jnp.arange(U, dtype=jnp.int32)
    skeys = jax.lax.sort(keys)

    b2 = b.reshape(1, L)

    out = pl.pallas_call(
        _body,
        grid=(nblk,),
        in_specs=[
            pl.BlockSpec((B, F), lambda k: (0, 0)),
            pl.BlockSpec((BLK, F), lambda k: (k, 0)),
            pl.BlockSpec((1, BLK), lambda k: (0, k)),
            pl.BlockSpec((U, F), lambda k: (0, 0)),
            pl.BlockSpec(memory_space=pltpu.SMEM),
        ],
        out_specs=pl.BlockSpec((B, BLK), lambda k: (0, k)),
        out_shape=jax.ShapeDtypeStruct((B, L), jnp.float32),
    )(x, W, b2, dict_val, skeys)
    return out
